# baseline (speedup 1.0000x reference)
import functools

import jax
import jax.numpy as jnp
from jax import lax
from jax.experimental import pallas as pl
from jax.experimental.pallas import tpu as pltpu
from jax.experimental.pallas import tpu_sc as plsc

E = 8
TOPK = 2
D_MODEL = 1024
D_FF = 4096
N = 2048
C = 512
FF_BLK = 2048
NFF = D_FF // FF_BLK

NW = 32
TPW = N // NW
CH = 32
LANES = 16


def _router_body(tok_ref, wg_ref, flat1_ref, flat2_ref, g1_ref, g2_ref):
    tok = tok_ref[...]
    wg = wg_ref[...]
    logits = jnp.dot(tok, wg, preferred_element_type=jnp.float32)
    m = jnp.max(logits, axis=1, keepdims=True)
    ex = jnp.exp(logits - m)
    probs = ex / jnp.sum(ex, axis=1, keepdims=True)

    col = jax.lax.broadcasted_iota(jnp.int32, (N, E), 1)
    big = jnp.int32(E)
    m1 = jnp.max(probs, axis=1, keepdims=True)
    a1 = jnp.min(jnp.where(probs == m1, col, big), axis=1, keepdims=True)
    p2 = jnp.where(col == a1, -1.0, probs)
    m2 = jnp.max(p2, axis=1, keepdims=True)
    a2 = jnp.min(jnp.where(p2 == m2, col, big), axis=1, keepdims=True)

    mask1 = (col == a1).astype(jnp.float32)
    mask2 = (col == a2).astype(jnp.float32)

    ri = jax.lax.broadcasted_iota(jnp.int32, (N, N), 0)
    ci = jax.lax.broadcasted_iota(jnp.int32, (N, N), 1)
    tril = (ci <= ri).astype(jnp.float32)
    cum1 = jnp.dot(tril, mask1, preferred_element_type=jnp.float32)
    cum2 = jnp.dot(tril, mask2, preferred_element_type=jnp.float32)

    pos1 = jnp.sum(cum1 * mask1, axis=1, keepdims=True) - 1.0
    c1 = jnp.sum(mask1, axis=0, keepdims=True)
    pos2 = (jnp.sum(cum2 * mask2, axis=1, keepdims=True) - 1.0
            + jnp.sum(c1 * mask2, axis=1, keepdims=True))
    pos1i = pos1.astype(jnp.int32)
    pos2i = pos2.astype(jnp.int32)

    keep1 = pos1i < C
    keep2 = pos2i < C
    flat1_ref[...] = jnp.where(keep1, a1 * C + pos1i, E * C)
    flat2_ref[...] = jnp.where(keep2, a2 * C + pos2i, E * C)
    g1 = jnp.where(keep1, m1, 0.0)
    g2 = jnp.where(keep2, m2, 0.0)
    denom = g1 + g2 + 1e-9
    g1_ref[...] = g1 / denom
    g2_ref[...] = g2 / denom


def _ffn_body(tok_ref, flat1_ref, flat2_ref, w1_ref, b1_ref, w2_ref, b2_ref,
              out_ref, eb_ref, acc_ref):
    e = pl.program_id(0)
    j = pl.program_id(1)

    @pl.when(j == 0)
    def _dispatch():
        slot = jax.lax.broadcasted_iota(jnp.int32, (C, N), 0) + e * C
        f1 = flat1_ref[...]
        f2 = flat2_ref[...]
        p = ((slot == f1) | (slot == f2)).astype(jnp.bfloat16)
        eb_ref[...] = jnp.dot(p, tok_ref[...],
                              preferred_element_type=jnp.float32
                              ).astype(jnp.bfloat16)

    h = jnp.dot(eb_ref[...], w1_ref[0].astype(jnp.bfloat16),
                preferred_element_type=jnp.float32) + b1_ref[0]
    h = jnp.where(h >= 0.0, h, 0.01 * h)
    part = jnp.dot(h.astype(jnp.bfloat16), w2_ref[0].astype(jnp.bfloat16),
                   preferred_element_type=jnp.float32)

    @pl.when(j == 0)
    def _init():
        acc_ref[...] = part

    @pl.when(j == NFF - 1)
    def _fin():
        out_ref[...] = acc_ref[...] + part + b2_ref[0]


def _sc_combine_body(eo_hbm, flat1_hbm, flat2_hbm, g1_hbm, g2_hbm, out_hbm,
                     idx1_v, idx2_v, g1_v, g2_v, rows1_v, rows2_v, sem):
    wid = lax.axis_index("s") * 2 + lax.axis_index("c")
    for cc in range(TPW // CH):
        base = wid * TPW + cc * CH
        pltpu.sync_copy(flat1_hbm.at[pl.ds(base, CH)], idx1_v)
        pltpu.sync_copy(flat2_hbm.at[pl.ds(base, CH)], idx2_v)
        pltpu.sync_copy(g1_hbm.at[pl.ds(base, CH)], g1_v)
        pltpu.sync_copy(g2_hbm.at[pl.ds(base, CH)], g2_v)
        for k in range(CH // LANES):
            sl = pl.ds(k * LANES, LANES)
            idx1_v[sl] = jnp.minimum(idx1_v[sl], E * C - 1)
            idx2_v[sl] = jnp.minimum(idx2_v[sl], E * C - 1)
        pltpu.async_copy(eo_hbm.at[idx1_v], rows1_v, sem).wait()
        pltpu.async_copy(eo_hbm.at[idx2_v], rows2_v, sem).wait()

        for q in range(CH // LANES):
            gvec1 = g1_v[pl.ds(q * LANES, LANES)]
            gvec2 = g2_v[pl.ds(q * LANES, LANES)]

            def tok_body(i, _, gvec1=gvec1, gvec2=gvec2, q=q):
                splat = jnp.full((LANES,), i, dtype=jnp.int32)
                gb1 = gvec1.at[splat].get(mode="promise_in_bounds")
                gb2 = gvec2.at[splat].get(mode="promise_in_bounds")
                t = q * LANES + i

                def chunk_body(k, _):
                    sl = pl.ds(k * LANES, LANES)
                    mixed = rows1_v[t, sl] * gb1 + rows2_v[t, sl] * gb2
                    rows1_v[t, sl] = mixed
                    return 0

                lax.fori_loop(0, D_MODEL // LANES, chunk_body, 0)
                return 0

            lax.fori_loop(0, LANES, tok_body, 0)
        pltpu.sync_copy(rows1_v, out_hbm.at[pl.ds(base, CH)])


@functools.partial(jax.jit, static_argnames=())
def kernel(x, wg, w1, b1, w2, b2):
    B, S, D = x.shape
    tok = x.reshape(N, D)
    tok_bf = tok.astype(jnp.bfloat16)

    flat1, flat2, g1, g2 = pl.pallas_call(
        _router_body,
        out_shape=(
            jax.ShapeDtypeStruct((N, 1), jnp.int32),
            jax.ShapeDtypeStruct((N, 1), jnp.int32),
            jax.ShapeDtypeStruct((N, 1), jnp.float32),
            jax.ShapeDtypeStruct((N, 1), jnp.float32),
        ),
    )(tok, wg)

    flat1_row = flat1.reshape(1, N)
    flat2_row = flat2.reshape(1, N)

    eo = pl.pallas_call(
        _ffn_body,
        grid=(E, NFF),
        in_specs=[
            pl.BlockSpec((N, D), lambda e, j: (0, 0)),
            pl.BlockSpec((1, N), lambda e, j: (0, 0)),
            pl.BlockSpec((1, N), lambda e, j: (0, 0)),
            pl.BlockSpec((1, D, FF_BLK), lambda e, j: (e, 0, j)),
            pl.BlockSpec((1, 1, FF_BLK), lambda e, j: (e, 0, j)),
            pl.BlockSpec((1, FF_BLK, D), lambda e, j: (e, j, 0)),
            pl.BlockSpec((1, 1, D), lambda e, j: (e, 0, 0)),
        ],
        out_specs=pl.BlockSpec((C, D), lambda e, j: (e, 0)),
        out_shape=jax.ShapeDtypeStruct((E * C, D), jnp.float32),
        scratch_shapes=[pltpu.VMEM((C, D), jnp.bfloat16),
                        pltpu.VMEM((C, D), jnp.float32)],
    )(tok_bf, flat1_row, flat2_row, w1, b1.reshape(E, 1, D_FF), w2,
      b2.reshape(E, 1, D))

    sc_combine = functools.partial(
        pl.kernel,
        out_type=jax.ShapeDtypeStruct((N, D), jnp.float32),
        mesh=plsc.VectorSubcoreMesh(core_axis_name="c", subcore_axis_name="s"),
        scratch_types=[
            pltpu.VMEM((CH,), jnp.int32),
            pltpu.VMEM((CH,), jnp.int32),
            pltpu.VMEM((CH,), jnp.float32),
            pltpu.VMEM((CH,), jnp.float32),
            pltpu.VMEM((CH, D_MODEL), jnp.float32),
            pltpu.VMEM((CH, D_MODEL), jnp.float32),
            pltpu.SemaphoreType.DMA,
        ],
    )(_sc_combine_body)

    out = sc_combine(eo, flat1.reshape(N), flat2.reshape(N),
                     g1.reshape(N), g2.reshape(N))

    return out.reshape(B, S, D)


# speedup vs baseline: 1.0708x; 1.0708x over previous
import functools

import jax
import jax.numpy as jnp
from jax import lax
from jax.experimental import pallas as pl
from jax.experimental.pallas import tpu as pltpu
from jax.experimental.pallas import tpu_sc as plsc

E = 8
TOPK = 2
D_MODEL = 1024
D_FF = 4096
N = 2048
C = 512
FF_BLK = 2048
NFF = D_FF // FF_BLK

NW = 32
TPW = N // NW
CH = 16
LANES = 16


def _router_body(tok_ref, wg_ref, flat1_ref, flat2_ref, g1_ref, g2_ref):
    tok = tok_ref[...]
    wg = wg_ref[...]
    logits = jnp.dot(tok, wg, preferred_element_type=jnp.float32)
    m = jnp.max(logits, axis=1, keepdims=True)
    ex = jnp.exp(logits - m)
    probs = ex / jnp.sum(ex, axis=1, keepdims=True)

    col = jax.lax.broadcasted_iota(jnp.int32, (N, E), 1)
    big = jnp.int32(E)
    m1 = jnp.max(probs, axis=1, keepdims=True)
    a1 = jnp.min(jnp.where(probs == m1, col, big), axis=1, keepdims=True)
    p2 = jnp.where(col == a1, -1.0, probs)
    m2 = jnp.max(p2, axis=1, keepdims=True)
    a2 = jnp.min(jnp.where(p2 == m2, col, big), axis=1, keepdims=True)

    mask1 = (col == a1).astype(jnp.float32)
    mask2 = (col == a2).astype(jnp.float32)

    ri = jax.lax.broadcasted_iota(jnp.int32, (N, N), 0)
    ci = jax.lax.broadcasted_iota(jnp.int32, (N, N), 1)
    tril = (ci <= ri).astype(jnp.float32)
    cum1 = jnp.dot(tril, mask1, preferred_element_type=jnp.float32)
    cum2 = jnp.dot(tril, mask2, preferred_element_type=jnp.float32)

    pos1 = jnp.sum(cum1 * mask1, axis=1, keepdims=True) - 1.0
    c1 = jnp.sum(mask1, axis=0, keepdims=True)
    pos2 = (jnp.sum(cum2 * mask2, axis=1, keepdims=True) - 1.0
            + jnp.sum(c1 * mask2, axis=1, keepdims=True))
    pos1i = pos1.astype(jnp.int32)
    pos2i = pos2.astype(jnp.int32)

    keep1 = pos1i < C
    keep2 = pos2i < C
    flat1_ref[...] = jnp.where(keep1, a1 * C + pos1i, E * C)
    flat2_ref[...] = jnp.where(keep2, a2 * C + pos2i, E * C)
    g1 = jnp.where(keep1, m1, 0.0)
    g2 = jnp.where(keep2, m2, 0.0)
    denom = g1 + g2 + 1e-9
    g1_ref[...] = g1 / denom
    g2_ref[...] = g2 / denom


def _ffn_body(tok_ref, flat1_ref, flat2_ref, w1_ref, b1_ref, w2_ref, b2_ref,
              out_ref, eb_ref, acc_ref):
    e = pl.program_id(0)
    j = pl.program_id(1)

    @pl.when(j == 0)
    def _dispatch():
        slot = jax.lax.broadcasted_iota(jnp.int32, (C, N), 0) + e * C
        f1 = flat1_ref[...]
        f2 = flat2_ref[...]
        p = ((slot == f1) | (slot == f2)).astype(jnp.bfloat16)
        eb_ref[...] = jnp.dot(p, tok_ref[...],
                              preferred_element_type=jnp.float32
                              ).astype(jnp.bfloat16)

    h = jnp.dot(eb_ref[...], w1_ref[0].astype(jnp.bfloat16),
                preferred_element_type=jnp.float32) + b1_ref[0]
    h = jnp.where(h >= 0.0, h, 0.01 * h)
    part = jnp.dot(h.astype(jnp.bfloat16), w2_ref[0].astype(jnp.bfloat16),
                   preferred_element_type=jnp.float32)

    @pl.when(j == 0)
    def _init():
        acc_ref[...] = part

    @pl.when(j == NFF - 1)
    def _fin():
        out_ref[...] = acc_ref[...] + part + b2_ref[0]


def _sc_combine_body(eo_hbm, flat1_hbm, flat2_hbm, g1_hbm, g2_hbm, out_hbm,
                     idx1_v, idx2_v, g1_v, g2_v, rows1_v, rows2_v,
                     sem_a, sem_b):
    wid = lax.axis_index("s") * 2 + lax.axis_index("c")
    nch = TPW // CH
    sems = (sem_a, sem_b)

    def _issue(cc):
        s = cc % 2
        base = wid * TPW + cc * CH
        pltpu.sync_copy(flat1_hbm.at[pl.ds(base, CH)], idx1_v.at[s])
        pltpu.sync_copy(flat2_hbm.at[pl.ds(base, CH)], idx2_v.at[s])
        pltpu.sync_copy(g1_hbm.at[pl.ds(base, CH)], g1_v.at[s])
        pltpu.sync_copy(g2_hbm.at[pl.ds(base, CH)], g2_v.at[s])
        idx1_v[s] = jnp.minimum(idx1_v[s], E * C - 1)
        idx2_v[s] = jnp.minimum(idx2_v[s], E * C - 1)
        c1 = pltpu.async_copy(eo_hbm.at[idx1_v.at[s]], rows1_v.at[s],
                              sems[s])
        c2 = pltpu.async_copy(eo_hbm.at[idx2_v.at[s]], rows2_v.at[s],
                              sems[s])
        return c1, c2

    pending = _issue(0)
    for cc in range(nch):
        s = cc % 2
        base = wid * TPW + cc * CH
        pending[0].wait()
        pending[1].wait()
        if cc + 1 < nch:
            pending = _issue(cc + 1)
        gvec1 = g1_v[s]
        gvec2 = g2_v[s]

        def tok_body(i, _, gvec1=gvec1, gvec2=gvec2, s=s):
            splat = jnp.full((LANES,), i, dtype=jnp.int32)
            gb1 = gvec1.at[splat].get(mode="promise_in_bounds")
            gb2 = gvec2.at[splat].get(mode="promise_in_bounds")
            for k in range(D_MODEL // LANES):
                sl = pl.ds(k * LANES, LANES)
                rows1_v[s, i, sl] = (rows1_v[s, i, sl] * gb1
                                     + rows2_v[s, i, sl] * gb2)
            return 0

        lax.fori_loop(0, CH, tok_body, 0)
        pltpu.sync_copy(rows1_v.at[s], out_hbm.at[pl.ds(base, CH)])


@functools.partial(jax.jit, static_argnames=())
def kernel(x, wg, w1, b1, w2, b2):
    B, S, D = x.shape
    tok = x.reshape(N, D)
    tok_bf = tok.astype(jnp.bfloat16)

    flat1, flat2, g1, g2 = pl.pallas_call(
        _router_body,
        out_shape=(
            jax.ShapeDtypeStruct((N, 1), jnp.int32),
            jax.ShapeDtypeStruct((N, 1), jnp.int32),
            jax.ShapeDtypeStruct((N, 1), jnp.float32),
            jax.ShapeDtypeStruct((N, 1), jnp.float32),
        ),
    )(tok, wg)

    flat1_row = flat1.reshape(1, N)
    flat2_row = flat2.reshape(1, N)

    eo = pl.pallas_call(
        _ffn_body,
        grid=(E, NFF),
        in_specs=[
            pl.BlockSpec((N, D), lambda e, j: (0, 0)),
            pl.BlockSpec((1, N), lambda e, j: (0, 0)),
            pl.BlockSpec((1, N), lambda e, j: (0, 0)),
            pl.BlockSpec((1, D, FF_BLK), lambda e, j: (e, 0, j)),
            pl.BlockSpec((1, 1, FF_BLK), lambda e, j: (e, 0, j)),
            pl.BlockSpec((1, FF_BLK, D), lambda e, j: (e, j, 0)),
            pl.BlockSpec((1, 1, D), lambda e, j: (e, 0, 0)),
        ],
        out_specs=pl.BlockSpec((C, D), lambda e, j: (e, 0)),
        out_shape=jax.ShapeDtypeStruct((E * C, D), jnp.float32),
        scratch_shapes=[pltpu.VMEM((C, D), jnp.bfloat16),
                        pltpu.VMEM((C, D), jnp.float32)],
    )(tok_bf, flat1_row, flat2_row, w1, b1.reshape(E, 1, D_FF), w2,
      b2.reshape(E, 1, D))

    sc_combine = functools.partial(
        pl.kernel,
        out_type=jax.ShapeDtypeStruct((N, D), jnp.float32),
        mesh=plsc.VectorSubcoreMesh(core_axis_name="c", subcore_axis_name="s"),
        scratch_types=[
            pltpu.VMEM((2, CH), jnp.int32),
            pltpu.VMEM((2, CH), jnp.int32),
            pltpu.VMEM((2, CH), jnp.float32),
            pltpu.VMEM((2, CH), jnp.float32),
            pltpu.VMEM((2, CH, D_MODEL), jnp.float32),
            pltpu.VMEM((2, CH, D_MODEL), jnp.float32),
            pltpu.SemaphoreType.DMA,
            pltpu.SemaphoreType.DMA,
        ],
    )(_sc_combine_body)

    out = sc_combine(eo, flat1.reshape(N), flat2.reshape(N),
                     g1.reshape(N), g2.reshape(N))

    return out.reshape(B, S, D)
